# baseline (speedup 1.0000x reference)
import functools

import jax
import jax.numpy as jnp
from jax import lax
from jax.experimental import pallas as pl
from jax.experimental.pallas import tpu as pltpu
from jax.experimental.pallas import tpu_sc as plsc

VOCAB = 1000000
DIM = 32
B = 4096
L = 50

_info = plsc.get_sparse_core_info()
_NC = _info.num_cores
_NS = _info.num_subcores
_NW = _NC * _NS
_BW = B // _NW
_NL = 5
_NJ = L // _NL


def _transpose_chunk(rows2, tile2, lane16):
    def body(g, carry):
        for u in range(4):
            r = g * 4 + u
            li = r >> 7
            bl = r & 127
            row0 = lane16 + li * 32
            col = jnp.full((16,), bl, jnp.int32)
            v0 = rows2[r, pl.ds(0, 16)]
            v1 = rows2[r, pl.ds(16, 16)]
            plsc.store_scatter(tile2, [row0, col], v0)
            plsc.store_scatter(tile2, [row0 + 16, col], v1)
        return carry
    lax.fori_loop(0, (_NL * _BW) // 4, body, 0)


def _gather_kernel(post_i, resp_i, wiki_i, table, post_o, resp_o, wiki_o,
                   idx_v, rows_a, rows_b, tile_a, tile_b, sem_g, sem_w,
                   sem_i):
    rows_v = (rows_a, rows_b)
    tile_v = (tile_a, tile_b)
    wid = lax.axis_index("s") * _NC + lax.axis_index("c")
    b0 = wid * _BW
    lane16 = lax.iota(jnp.int32, 16)
    streams = ((post_i, post_o), (resp_i, resp_o), (wiki_i, wiki_o))
    jobs = [(s, l0) for s in range(3) for l0 in range(0, L, _NL)]
    n = len(jobs)

    def transform_idx(b):
        def body(t, carry):
            li = t >> 3
            x = t & 7
            sl = idx_v[b, li, pl.ds(x * 16, 16)]
            q = ((sl >= _G).astype(jnp.int32)
                 + (sl >= 2 * _G).astype(jnp.int32)
                 + (sl >= 3 * _G).astype(jnp.int32))
            idx_v[b, li, pl.ds(x * 16, 16)] = sl * 4 - q * (4 * _G - 1)
            return carry
        lax.fori_loop(0, (_NL * _BW) // 16, body, 0)

    def start_idx(j, b):
        s, l0 = jobs[j]
        return pltpu.async_copy(
            streams[s][0].at[pl.ds(l0, _NL), pl.ds(b0, _BW)],
            idx_v.at[b], sem_i)

    def fire_gathers(j, b):
        return [pltpu.async_copy(table.at[idx_v.at[b, li]],
                                 rows_v[b].at[pl.ds(li * _BW, _BW)], sem_g)
                for li in range(_NL)]

    def fire_writebacks(j, b):
        s, l0 = jobs[j]
        out = streams[s][1]
        tile = tile_v[b]

        def body(t, carry):
            pltpu.async_copy(tile.at[pl.ds(t * 8, 8), pl.ds(0, _BW)],
                             out.at[l0 + (t >> 2), t & 3, wid], sem_w)
            return carry
        lax.fori_loop(0, _NL * 4, body, 0)

    def drain_writebacks(k):
        def body(t, carry):
            pltpu.make_async_copy(
                tile_a.at[pl.ds(0, 8), pl.ds(0, _BW)],
                post_o.at[0, 0, wid], sem_w).wait()
            return carry
        lax.fori_loop(0, k, body, 0)

    gat = [None, None]
    idxc = [None, None]
    idxc[0] = start_idx(0, 0)
    idxc[0].wait()
    transform_idx(0)
    gat[0] = fire_gathers(0, 0)
    idxc[1] = start_idx(1, 1)
    for j in range(1, n):
        b = j & 1
        pb = 1 - b
        if j >= 2:
            drain_writebacks(_NL * 4)
        for cp in gat[pb]:
            cp.wait()
        idxc[b].wait()
        transform_idx(b)
        gat[b] = fire_gathers(j, b)
        if j + 1 < n:
            idxc[pb] = start_idx(j + 1, pb)
        _transpose_chunk(rows_v[pb], tile_v[pb], lane16)
        fire_writebacks(j - 1, pb)
    lb = (n - 1) & 1
    for cp in gat[lb]:
        cp.wait()
    _transpose_chunk(rows_v[lb], tile_v[lb], lane16)
    fire_writebacks(n - 1, lb)
    drain_writebacks(2 * _NL * 4)


_G = 250112


def _repack_block(s0, s1, s2, s3, out_ref):
    out_ref[:, 0:32] = s0[...].T
    out_ref[:, 32:64] = s1[...].T
    out_ref[:, 64:96] = s2[...].T
    out_ref[:, 96:128] = s3[...].T


@jax.jit
def _repack(table_t):
    n_blk = _G // 128
    specs = [
        pl.BlockSpec((32, 128), (lambda i, q=q: (0, n_blk * q + i)))
        for q in range(4)
    ]
    return pl.pallas_call(
        _repack_block,
        grid=(n_blk,),
        in_specs=specs,
        out_specs=pl.BlockSpec((128, 128), lambda i: (i, 0)),
        out_shape=jax.ShapeDtypeStruct((_G, 128), jnp.float32),
    )(table_t, table_t, table_t, table_t)


@jax.jit
def _embed3(post_t, resp_t, wiki_t, table):
    mesh = plsc.VectorSubcoreMesh(core_axis_name="c", subcore_axis_name="s")
    out = jax.ShapeDtypeStruct((L, 4, _NW, 8, _BW), jnp.float32)
    return pl.kernel(
        _gather_kernel,
        mesh=mesh,
        out_type=(out, out, out),
        scratch_types=[
            pltpu.VMEM((2, _NL, _BW), jnp.int32),
            pltpu.VMEM((_NL * _BW, DIM), jnp.float32),
            pltpu.VMEM((_NL * _BW, DIM), jnp.float32),
            pltpu.VMEM((_NL * 4 * 8, 129), jnp.float32),
            pltpu.VMEM((_NL * 4 * 8, 129), jnp.float32),
            pltpu.SemaphoreType.DMA,
            pltpu.SemaphoreType.DMA,
            pltpu.SemaphoreType.DMA,
        ],
        compiler_params=pltpu.CompilerParams(use_tc_tiling_on_sc=False,
                                             needs_layout_passes=False),
    )(post_t, resp_t, wiki_t, table)


def kernel(post, resp, wiki, table):
    lin_table = _repack(jnp.transpose(table)).reshape(4 * _G, DIM)
    outs = _embed3(jnp.transpose(post), jnp.transpose(resp),
                   jnp.transpose(wiki), lin_table)
    return tuple(
        o.transpose(2, 4, 0, 1, 3).reshape(B, L, DIM) for o in outs)


# speedup vs baseline: 1.0005x; 1.0005x over previous
import functools

import jax
import jax.numpy as jnp
from jax import lax
from jax.experimental import pallas as pl
from jax.experimental.pallas import tpu as pltpu
from jax.experimental.pallas import tpu_sc as plsc

VOCAB = 1000000
DIM = 32
B = 4096
L = 50

_info = plsc.get_sparse_core_info()
_NC = _info.num_cores
_NS = _info.num_subcores
_NW = _NC * _NS
_BW = B // _NW
_NL = 5
_NJ = L // _NL


def _transpose_chunk(rows2, tile2, lane16):
    def body(g, carry):
        for u in range(4):
            r = g * 4 + u
            li = r >> 7
            bl = r & 127
            row0 = lane16 + li * 32
            col = jnp.full((16,), bl, jnp.int32)
            v0 = rows2[r, pl.ds(0, 16)]
            v1 = rows2[r, pl.ds(16, 16)]
            plsc.store_scatter(tile2, [row0, col], v0)
            plsc.store_scatter(tile2, [row0 + 16, col], v1)
        return carry
    lax.fori_loop(0, (_NL * _BW) // 4, body, 0)


def _gather_kernel(post_i, resp_i, wiki_i, table, post_o, resp_o, wiki_o,
                   idx_v, rows_a, rows_b, tile_a, tile_b, sem_g, sem_w,
                   sem_i):
    rows_v = (rows_a, rows_b)
    tile_v = (tile_a, tile_b)
    wid = lax.axis_index("s") * _NC + lax.axis_index("c")
    b0 = wid * _BW
    lane16 = lax.iota(jnp.int32, 16)
    streams = ((post_i, post_o), (resp_i, resp_o), (wiki_i, wiki_o))
    jobs = [(s, l0) for s in range(3) for l0 in range(0, L, _NL)]
    n = len(jobs)

    def transform_idx(b):
        def body(t, carry):
            li = t >> 3
            x = t & 7
            sl = idx_v[b, li, pl.ds(x * 16, 16)]
            q = ((sl >= _G).astype(jnp.int32)
                 + (sl >= 2 * _G).astype(jnp.int32)
                 + (sl >= 3 * _G).astype(jnp.int32))
            idx_v[b, li, pl.ds(x * 16, 16)] = sl * 4 - q * (4 * _G - 1)
            return carry
        lax.fori_loop(0, (_NL * _BW) // 16, body, 0)

    def start_idx(j, b):
        s, l0 = jobs[j]
        return pltpu.async_copy(
            streams[s][0].at[pl.ds(l0, _NL), pl.ds(b0, _BW)],
            idx_v.at[b], sem_i)

    def fire_gathers(j, b):
        return [pltpu.async_copy(table.at[idx_v.at[b, li]],
                                 rows_v[b].at[pl.ds(li * _BW, _BW)], sem_g)
                for li in range(_NL)]

    def fire_writebacks(j, b):
        s, l0 = jobs[j]
        out = streams[s][1]
        tile = tile_v[b]

        def body(t, carry):
            pltpu.async_copy(tile.at[pl.ds(t * 8, 8), pl.ds(0, _BW)],
                             out.at[l0 + (t >> 2), t & 3, wid], sem_w)
            return carry
        lax.fori_loop(0, _NL * 4, body, 0)

    def drain_writebacks(k):
        def body(t, carry):
            pltpu.make_async_copy(
                tile_a.at[pl.ds(0, 8), pl.ds(0, _BW)],
                post_o.at[0, 0, wid], sem_w).wait()
            return carry
        lax.fori_loop(0, k, body, 0)

    gat = [None, None]
    idxc = [None, None]
    idxc[0] = start_idx(0, 0)
    idxc[0].wait()
    transform_idx(0)
    gat[0] = fire_gathers(0, 0)
    idxc[1] = start_idx(1, 1)
    for j in range(1, n):
        b = j & 1
        pb = 1 - b
        if j >= 2:
            drain_writebacks(_NL * 4)
        for cp in gat[pb]:
            cp.wait()
        idxc[b].wait()
        transform_idx(b)
        gat[b] = fire_gathers(j, b)
        if j + 1 < n:
            idxc[pb] = start_idx(j + 1, pb)
        _transpose_chunk(rows_v[pb], tile_v[pb], lane16)
        fire_writebacks(j - 1, pb)
    lb = (n - 1) & 1
    for cp in gat[lb]:
        cp.wait()
    _transpose_chunk(rows_v[lb], tile_v[lb], lane16)
    fire_writebacks(n - 1, lb)
    drain_writebacks(2 * _NL * 4)


_G = 250112


def _repack_transpose(src_v, dst_v, lane16):
    def body(g, carry):
        for u in range(4):
            t = g * 4 + u
            d = t >> 3
            x = t & 7
            v = src_v[d, pl.ds(x * 16, 16)]
            plsc.store_scatter(dst_v, [lane16 + x * 16,
                                       jnp.full((16,), d, jnp.int32)], v)
        return carry
    lax.fori_loop(0, 64, body, 0)


def _repack_kernel(tbl_t, out_r, src_a, src_b, dst_a, dst_b, sem_i, sem_o):
    wid = lax.axis_index("s") * _NC + lax.axis_index("c")
    lane16 = lax.iota(jnp.int32, 16)
    last = VOCAB - 128

    def do_block(c_idx, src_v, dst_v, fire_in_only, base_out):
        base = jnp.minimum(c_idx * 128, last)
        if fire_in_only:
            return pltpu.async_copy(tbl_t.at[:, pl.ds(base, 128)], src_v,
                                    sem_i)
        q = ((base >= _G).astype(jnp.int32)
             + (base >= 2 * _G).astype(jnp.int32)
             + (base >= 3 * _G).astype(jnp.int32))
        p0 = base - q * _G
        _repack_transpose(src_v, dst_v, lane16)
        pltpu.async_copy(dst_v.at[pl.ds(0, 128), pl.ds(0, 32)],
                         out_r.at[pl.ds(p0, 128), pl.ds(32 * q, 32)], sem_o)

    def drain_out():
        pltpu.make_async_copy(dst_a.at[pl.ds(0, 128), pl.ds(0, 32)],
                              out_r.at[pl.ds(0, 128), pl.ds(0, 32)],
                              sem_o).wait()

    def body(k2, carry):
        c0 = wid + 64 * k2
        c1 = c0 + 32
        cp0 = do_block(c0, src_a, dst_a, True, None)
        cp1 = do_block(c1, src_b, dst_b, True, None)

        @pl.when(k2 > 0)
        def _():
            drain_out()
            drain_out()
        cp0.wait()
        do_block(c0, src_a, dst_a, False, None)
        cp1.wait()
        do_block(c1, src_b, dst_b, False, None)
        return carry
    lax.fori_loop(0, 123, body, 0)
    drain_out()
    drain_out()


@jax.jit
def _repack(table_t):
    mesh = plsc.VectorSubcoreMesh(core_axis_name="c", subcore_axis_name="s")
    return pl.kernel(
        _repack_kernel,
        mesh=mesh,
        out_type=jax.ShapeDtypeStruct((_G, 128), jnp.float32),
        scratch_types=[
            pltpu.VMEM((32, 128), jnp.float32),
            pltpu.VMEM((32, 128), jnp.float32),
            pltpu.VMEM((128, 33), jnp.float32),
            pltpu.VMEM((128, 33), jnp.float32),
            pltpu.SemaphoreType.DMA,
            pltpu.SemaphoreType.DMA,
        ],
        compiler_params=pltpu.CompilerParams(use_tc_tiling_on_sc=True),
    )(table_t)


def _gather_kernel(post_i, resp_i, wiki_i, table, post_o, resp_o, wiki_o,
                   idx_v, rows_a, rows_b, tile_a, tile_b, sem_g, sem_w,
                   sem_i):
    rows_v = (rows_a, rows_b)
    tile_v = (tile_a, tile_b)
    wid = lax.axis_index("s") * _NC + lax.axis_index("c")
    b0 = wid * _BW
    lane16 = lax.iota(jnp.int32, 16)
    streams = ((post_i, post_o), (resp_i, resp_o), (wiki_i, wiki_o))
    jobs = [(s, l0) for s in range(3) for l0 in range(0, L, _NL)]
    n = len(jobs)

    def transform_idx(b):
        def body(t, carry):
            li = t >> 3
            x = t & 7
            sl = idx_v[b, li, pl.ds(x * 16, 16)]
            q = ((sl >= _G).astype(jnp.int32)
                 + (sl >= 2 * _G).astype(jnp.int32)
                 + (sl >= 3 * _G).astype(jnp.int32))
            idx_v[b, li, pl.ds(x * 16, 16)] = sl * 4 - q * (4 * _G - 1)
            return carry
        lax.fori_loop(0, (_NL * _BW) // 16, body, 0)

    def start_idx(j, b):
        s, l0 = jobs[j]
        return pltpu.async_copy(
            streams[s][0].at[pl.ds(l0, _NL), pl.ds(b0, _BW)],
            idx_v.at[b], sem_i)

    def fire_gathers(j, b):
        return [pltpu.async_copy(table.at[idx_v.at[b, li]],
                                 rows_v[b].at[pl.ds(li * _BW, _BW)], sem_g)
                for li in range(_NL)]

    def fire_writebacks(j, b):
        s, l0 = jobs[j]
        out = streams[s][1]
        tile = tile_v[b]

        def body(t, carry):
            pltpu.async_copy(tile.at[pl.ds(t * 8, 8), pl.ds(0, _BW)],
                             out.at[l0 + (t >> 2), t & 3, wid], sem_w)
            return carry
        lax.fori_loop(0, _NL * 4, body, 0)

    def drain_writebacks(k):
        def body(t, carry):
            pltpu.make_async_copy(
                tile_a.at[pl.ds(0, 8), pl.ds(0, _BW)],
                post_o.at[0, 0, wid], sem_w).wait()
            return carry
        lax.fori_loop(0, k, body, 0)

    gat = [None, None]
    idxc = [None, None]
    idxc[0] = start_idx(0, 0)
    idxc[0].wait()
    transform_idx(0)
    gat[0] = fire_gathers(0, 0)
    idxc[1] = start_idx(1, 1)
    for j in range(1, n):
        b = j & 1
        pb = 1 - b
        if j >= 2:
            drain_writebacks(_NL * 4)
        for cp in gat[pb]:
            cp.wait()
        idxc[b].wait()
        transform_idx(b)
        gat[b] = fire_gathers(j, b)
        if j + 1 < n:
            idxc[pb] = start_idx(j + 1, pb)
        _transpose_chunk(rows_v[pb], tile_v[pb], lane16)
        fire_writebacks(j - 1, pb)
    lb = (n - 1) & 1
    for cp in gat[lb]:
        cp.wait()
    _transpose_chunk(rows_v[lb], tile_v[lb], lane16)
    fire_writebacks(n - 1, lb)
    drain_writebacks(2 * _NL * 4)


_G = 250112


def _repack_block(s0, s1, s2, s3, out_ref):
    out_ref[:, 0:32] = s0[...].T
    out_ref[:, 32:64] = s1[...].T
    out_ref[:, 64:96] = s2[...].T
    out_ref[:, 96:128] = s3[...].T


@jax.jit
def _repack(table_t):
    n_blk = _G // 128
    specs = [
        pl.BlockSpec((32, 128), (lambda i, q=q: (0, n_blk * q + i)))
        for q in range(4)
    ]
    return pl.pallas_call(
        _repack_block,
        grid=(n_blk,),
        in_specs=specs,
        out_specs=pl.BlockSpec((128, 128), lambda i: (i, 0)),
        out_shape=jax.ShapeDtypeStruct((_G, 128), jnp.float32),
    )(table_t, table_t, table_t, table_t)


@jax.jit
def _embed3(post_t, resp_t, wiki_t, table):
    mesh = plsc.VectorSubcoreMesh(core_axis_name="c", subcore_axis_name="s")
    out = jax.ShapeDtypeStruct((L, 4, _NW, 8, _BW), jnp.float32)
    return pl.kernel(
        _gather_kernel,
        mesh=mesh,
        out_type=(out, out, out),
        scratch_types=[
            pltpu.VMEM((2, _NL, _BW), jnp.int32),
            pltpu.VMEM((_NL * _BW, DIM), jnp.float32),
            pltpu.VMEM((_NL * _BW, DIM), jnp.float32),
            pltpu.VMEM((_NL * 4 * 8, 129), jnp.float32),
            pltpu.VMEM((_NL * 4 * 8, 129), jnp.float32),
            pltpu.SemaphoreType.DMA,
            pltpu.SemaphoreType.DMA,
            pltpu.SemaphoreType.DMA,
        ],
        compiler_params=pltpu.CompilerParams(use_tc_tiling_on_sc=False,
                                             needs_layout_passes=False),
    )(post_t, resp_t, wiki_t, table)


def kernel(post, resp, wiki, table):
    lin_table = _repack(jnp.transpose(table)).reshape(4 * _G, DIM)
    outs = _embed3(jnp.transpose(post), jnp.transpose(resp),
                   jnp.transpose(wiki), lin_table)
    return tuple(
        o.transpose(2, 4, 0, 1, 3).reshape(B, L, DIM) for o in outs)


# speedup vs baseline: 1.5838x; 1.5830x over previous
import functools

import jax
import jax.numpy as jnp
from jax import lax
from jax.experimental import pallas as pl
from jax.experimental.pallas import tpu as pltpu
from jax.experimental.pallas import tpu_sc as plsc

VOCAB = 1000000
DIM = 32
B = 4096
L = 50

_info = plsc.get_sparse_core_info()
_NC = _info.num_cores
_NS = _info.num_subcores
_NW = _NC * _NS
_BW = B // _NW
_NL = 5
_G = 250112


def _repack_transpose(src_v, dst_v, lane16, q, shift=None):
    def body(g, carry):
        for u in range(4):
            t = g * 4 + u
            d = t >> 3
            x = t & 7
            v = src_v[d, pl.ds(x * 16, 16)]
            col = jnp.full((16,), 32 * q + d, jnp.int32)
            row = lane16 + x * 16
            if shift is None:
                plsc.store_scatter(dst_v, [row, col], v)
            else:
                row = row - shift
                plsc.store_scatter(dst_v, [row, col], v, mask=row >= 0)
        return carry
    lax.fori_loop(0, 64, body, 0)


def _repack_kernel(tbl_t, out_r,
                   sa0, sa1, sa2, sa3, sb0, sb1, sb2, sb3,
                   dst_a, dst_b, sem_i, sem_o):
    wid = lax.axis_index("s") * _NC + lax.axis_index("c")
    lane16 = lax.iota(jnp.int32, 16)
    srcs_a = (sa0, sa1, sa2, sa3)
    srcs_b = (sb0, sb1, sb2, sb3)
    nblk = _G // 128
    lastc = nblk - 1

    def fire_in(k, srcs):
        c = jnp.minimum(wid + 32 * k, lastc)
        cps = []
        for q in range(4):
            col = pl.multiple_of(
                jnp.minimum(q * _G + c * 128, VOCAB - 128), 128)
            cps.append(pltpu.async_copy(tbl_t.at[:, pl.ds(col, 128)],
                                        srcs[q], sem_i))
        return cps

    def finish(k, srcs, dst_v):
        c = jnp.minimum(wid + 32 * k, lastc)
        for q in range(4):
            shift = None
            if q == 3:
                col0 = 3 * _G + c * 128
                shift = col0 - jnp.minimum(col0, VOCAB - 128)
            _repack_transpose(srcs[q], dst_v, lane16, q, shift)
        p0 = pl.multiple_of(c * 128, 128)
        pltpu.async_copy(dst_v.at[pl.ds(0, 128), pl.ds(0, 128)],
                         out_r.at[pl.ds(p0, 128), :], sem_o)

    def drain_out():
        pltpu.make_async_copy(dst_a.at[pl.ds(0, 128), pl.ds(0, 128)],
                              out_r.at[pl.ds(0, 128), :], sem_o).wait()

    def body(k2, carry):
        cps_a = fire_in(2 * k2, srcs_a)
        cps_b = fire_in(2 * k2 + 1, srcs_b)

        @pl.when(k2 > 0)
        def _():
            drain_out()
            drain_out()
        for cp in cps_a:
            cp.wait()
        finish(2 * k2, srcs_a, dst_a)
        for cp in cps_b:
            cp.wait()
        finish(2 * k2 + 1, srcs_b, dst_b)
        return carry
    lax.fori_loop(0, 31, body, 0)
    drain_out()
    drain_out()


@jax.jit
def _repack(table_t):
    mesh = plsc.VectorSubcoreMesh(core_axis_name="c", subcore_axis_name="s")
    src = pltpu.VMEM((32, 128), jnp.float32)
    return pl.kernel(
        _repack_kernel,
        mesh=mesh,
        out_type=jax.ShapeDtypeStruct((_G, 128), jnp.float32),
        scratch_types=[
            src, src, src, src, src, src, src, src,
            pltpu.VMEM((128, 129), jnp.float32),
            pltpu.VMEM((128, 129), jnp.float32),
            pltpu.SemaphoreType.DMA,
            pltpu.SemaphoreType.DMA,
        ],
        compiler_params=pltpu.CompilerParams(use_tc_tiling_on_sc=True,
                                             needs_layout_passes=False),
    )(table_t)


def _transpose_chunk(rows2, tile2, lane16):
    def body(g, carry):
        for u in range(4):
            r = g * 4 + u
            li = r >> 7
            bl = r & 127
            row0 = lane16 + li * 32
            col = jnp.full((16,), bl, jnp.int32)
            v0 = rows2[r, pl.ds(0, 16)]
            v1 = rows2[r, pl.ds(16, 16)]
            plsc.store_scatter(tile2, [row0, col], v0)
            plsc.store_scatter(tile2, [row0 + 16, col], v1)
        return carry
    lax.fori_loop(0, (_NL * _BW) // 4, body, 0)


def _gather_kernel(post_i, resp_i, wiki_i, table, post_o, resp_o, wiki_o,
                   idx_v, rows_a, rows_b, tile_a, tile_b, sem_g, sem_w,
                   sem_i):
    rows_v = (rows_a, rows_b)
    tile_v = (tile_a, tile_b)
    wid = lax.axis_index("s") * _NC + lax.axis_index("c")
    b0 = wid * _BW
    lane16 = lax.iota(jnp.int32, 16)
    streams = ((post_i, post_o), (resp_i, resp_o), (wiki_i, wiki_o))
    jobs = [(s, l0) for s in range(3) for l0 in range(0, L, _NL)]
    n = len(jobs)

    def transform_idx(b):
        def body(t, carry):
            li = t >> 3
            x = t & 7
            sl = idx_v[b, li, pl.ds(x * 16, 16)]
            q = ((sl >= _G).astype(jnp.int32)
                 + (sl >= 2 * _G).astype(jnp.int32)
                 + (sl >= 3 * _G).astype(jnp.int32))
            idx_v[b, li, pl.ds(x * 16, 16)] = sl * 4 - q * (4 * _G - 1)
            return carry
        lax.fori_loop(0, (_NL * _BW) // 16, body, 0)

    def start_idx(j, b):
        s, l0 = jobs[j]
        return pltpu.async_copy(
            streams[s][0].at[pl.ds(l0, _NL), pl.ds(b0, _BW)],
            idx_v.at[b], sem_i)

    def fire_gathers(j, b):
        return [pltpu.async_copy(table.at[idx_v.at[b, li]],
                                 rows_v[b].at[pl.ds(li * _BW, _BW)], sem_g)
                for li in range(_NL)]

    def fire_writebacks(j, b):
        s, l0 = jobs[j]
        out = streams[s][1]
        tile = tile_v[b]

        def body(t, carry):
            pltpu.async_copy(tile.at[pl.ds(t * 8, 8), pl.ds(0, _BW)],
                             out.at[l0 + (t >> 2), t & 3, wid], sem_w)
            return carry
        lax.fori_loop(0, _NL * 4, body, 0)

    def drain_writebacks(k):
        def body(t, carry):
            pltpu.make_async_copy(
                tile_a.at[pl.ds(0, 8), pl.ds(0, _BW)],
                post_o.at[0, 0, wid], sem_w).wait()
            return carry
        lax.fori_loop(0, k, body, 0)

    gat = [None, None]
    idxc = [None, None]
    idxc[0] = start_idx(0, 0)
    idxc[0].wait()
    transform_idx(0)
    gat[0] = fire_gathers(0, 0)
    idxc[1] = start_idx(1, 1)
    for j in range(1, n):
        b = j & 1
        pb = 1 - b
        if j >= 2:
            drain_writebacks(_NL * 4)
        for cp in gat[pb]:
            cp.wait()
        idxc[b].wait()
        transform_idx(b)
        gat[b] = fire_gathers(j, b)
        if j + 1 < n:
            idxc[pb] = start_idx(j + 1, pb)
        _transpose_chunk(rows_v[pb], tile_v[pb], lane16)
        fire_writebacks(j - 1, pb)
    lb = (n - 1) & 1
    for cp in gat[lb]:
        cp.wait()
    _transpose_chunk(rows_v[lb], tile_v[lb], lane16)
    fire_writebacks(n - 1, lb)
    drain_writebacks(2 * _NL * 4)


@jax.jit
def _embed3(post_t, resp_t, wiki_t, table):
    mesh = plsc.VectorSubcoreMesh(core_axis_name="c", subcore_axis_name="s")
    out = jax.ShapeDtypeStruct((L, 4, _NW, 8, _BW), jnp.float32)
    return pl.kernel(
        _gather_kernel,
        mesh=mesh,
        out_type=(out, out, out),
        scratch_types=[
            pltpu.VMEM((2, _NL, _BW), jnp.int32),
            pltpu.VMEM((_NL * _BW, DIM), jnp.float32),
            pltpu.VMEM((_NL * _BW, DIM), jnp.float32),
            pltpu.VMEM((_NL * 4 * 8, 129), jnp.float32),
            pltpu.VMEM((_NL * 4 * 8, 129), jnp.float32),
            pltpu.SemaphoreType.DMA,
            pltpu.SemaphoreType.DMA,
            pltpu.SemaphoreType.DMA,
        ],
        compiler_params=pltpu.CompilerParams(use_tc_tiling_on_sc=False,
                                             needs_layout_passes=False),
    )(post_t, resp_t, wiki_t, table)


def kernel(post, resp, wiki, table):
    packed = _repack(jnp.transpose(table))
    tail = VOCAB & 127
    p0 = (VOCAB - 3 * _G) - tail
    packed = jax.lax.dynamic_update_slice(
        packed, table[VOCAB - tail:, :], (p0, 3 * DIM))
    lin_table = packed.reshape(4 * _G, DIM)
    outs = _embed3(jnp.transpose(post), jnp.transpose(resp),
                   jnp.transpose(wiki), lin_table)
    return tuple(
        o.transpose(2, 4, 0, 1, 3).reshape(B, L, DIM) for o in outs)


# speedup vs baseline: 2.0963x; 1.3236x over previous
import functools

import jax
import jax.numpy as jnp
from jax import lax
from jax.experimental import pallas as pl
from jax.experimental.pallas import tpu as pltpu
from jax.experimental.pallas import tpu_sc as plsc

VOCAB = 1000000
DIM = 32
B = 4096
L = 50

_info = plsc.get_sparse_core_info()
_NC = _info.num_cores
_NS = _info.num_subcores
_NW = _NC * _NS
_BW = B // _NW
_NL = 5


def _transpose_chunk(rows2, tile2, lane16):
    def body(g, carry):
        for u in range(4):
            r = g * 4 + u
            li = r >> 7
            bl = r & 127
            row0 = lane16 + li * 32
            col = jnp.full((16,), bl, jnp.int32)
            v0 = rows2[r, pl.ds(0, 16)]
            v1 = rows2[r, pl.ds(16, 16)]
            plsc.store_scatter(tile2, [row0, col], v0)
            plsc.store_scatter(tile2, [row0 + 16, col], v1)
        return carry
    lax.fori_loop(0, (_NL * _BW) // 4, body, 0)


def _gather_kernel(post_i, resp_i, wiki_i, table, post_o, resp_o, wiki_o,
                   idx_v, rows_a, rows_b, tile_a, tile_b, sem_g, sem_w,
                   sem_i):
    rows_v = (rows_a, rows_b)
    tile_v = (tile_a, tile_b)
    wid = lax.axis_index("s") * _NC + lax.axis_index("c")
    b0 = wid * _BW
    lane16 = lax.iota(jnp.int32, 16)
    streams = ((post_i, post_o), (resp_i, resp_o), (wiki_i, wiki_o))
    jobs = [(s, l0) for s in range(3) for l0 in range(0, L, _NL)]
    n = len(jobs)

    def start_idx(j, b):
        s, l0 = jobs[j]
        return pltpu.async_copy(
            streams[s][0].at[pl.ds(l0, _NL), pl.ds(b0, _BW)],
            idx_v.at[b], sem_i)

    def fire_gathers(j, b):
        return [pltpu.async_copy(table.at[idx_v.at[b, li]],
                                 rows_v[b].at[pl.ds(li * _BW, _BW)], sem_g)
                for li in range(_NL)]

    def fire_writebacks(j, b):
        s, l0 = jobs[j]
        out = streams[s][1]
        tile = tile_v[b]

        def body(t, carry):
            pltpu.async_copy(tile.at[pl.ds(t * 8, 8), pl.ds(0, _BW)],
                             out.at[l0 + (t >> 2), t & 3, wid], sem_w)
            return carry
        lax.fori_loop(0, _NL * 4, body, 0)

    def drain_writebacks(k):
        def body(t, carry):
            pltpu.make_async_copy(
                tile_a.at[pl.ds(0, 8), pl.ds(0, _BW)],
                post_o.at[0, 0, wid], sem_w).wait()
            return carry
        lax.fori_loop(0, k, body, 0)

    gat = [None, None]
    idxc = [None, None]
    idxc[0] = start_idx(0, 0)
    idxc[0].wait()
    gat[0] = fire_gathers(0, 0)
    idxc[1] = start_idx(1, 1)
    for j in range(1, n):
        b = j & 1
        pb = 1 - b
        if j >= 2:
            drain_writebacks(_NL * 4)
        for cp in gat[pb]:
            cp.wait()
        idxc[b].wait()
        gat[b] = fire_gathers(j, b)
        if j + 1 < n:
            idxc[pb] = start_idx(j + 1, pb)
        _transpose_chunk(rows_v[pb], tile_v[pb], lane16)
        fire_writebacks(j - 1, pb)
    lb = (n - 1) & 1
    for cp in gat[lb]:
        cp.wait()
    _transpose_chunk(rows_v[lb], tile_v[lb], lane16)
    fire_writebacks(n - 1, lb)
    drain_writebacks(2 * _NL * 4)


@jax.jit
def _embed3(post_t, resp_t, wiki_t, table):
    mesh = plsc.VectorSubcoreMesh(core_axis_name="c", subcore_axis_name="s")
    out = jax.ShapeDtypeStruct((L, 4, _NW, 8, _BW), jnp.float32)
    return pl.kernel(
        _gather_kernel,
        mesh=mesh,
        out_type=(out, out, out),
        scratch_types=[
            pltpu.VMEM((2, _NL, _BW), jnp.int32),
            pltpu.VMEM((_NL * _BW, DIM), jnp.float32),
            pltpu.VMEM((_NL * _BW, DIM), jnp.float32),
            pltpu.VMEM((_NL * 4 * 8, 129), jnp.float32),
            pltpu.VMEM((_NL * 4 * 8, 129), jnp.float32),
            pltpu.SemaphoreType.DMA,
            pltpu.SemaphoreType.DMA,
            pltpu.SemaphoreType.DMA,
        ],
        compiler_params=pltpu.CompilerParams(use_tc_tiling_on_sc=False,
                                             needs_layout_passes=False),
    )(post_t, resp_t, wiki_t, table)


def kernel(post, resp, wiki, table):
    outs = _embed3(jnp.transpose(post), jnp.transpose(resp),
                   jnp.transpose(wiki), table)
    return tuple(
        o.transpose(2, 4, 0, 1, 3).reshape(B, L, DIM) for o in outs)
